# baseline (speedup 1.0000x reference)
import functools

import jax
import jax.numpy as jnp
from jax.experimental import pallas as pl
from jax.experimental.pallas import tpu as pltpu

_N_EXPERTS = 64
_TOP_K = 2
_D_IN = 768
_D_OUT = 768
_B = 4
_S = 2048
_LOSS_COEF = 0.01

_POOL_BLK = 512
_MM_BLK = 512


def _gating_body(x_ref, wg_ref, pi_ref, pf_ref, loss_ref, acc_ref):
    i = pl.program_id(0)
    n = pl.num_programs(0)

    @pl.when(i == 0)
    def _init():
        acc_ref[...] = jnp.zeros_like(acc_ref)

    acc_ref[...] += jnp.sum(x_ref[...], axis=1)

    @pl.when(i == n - 1)
    def _finish():
        pooled = acc_ref[...] * (1.0 / _S)
        logits = jax.lax.dot_general(
            pooled, wg_ref[...], (((1,), (0,)), ((), ())),
            preferred_element_type=jnp.float32)
        iota = jax.lax.broadcasted_iota(jnp.int32, (_B, _N_EXPERTS), 1)
        m1 = jnp.max(logits, axis=1, keepdims=True)
        a1 = jnp.min(jnp.where(logits == m1, iota, _N_EXPERTS), axis=1,
                     keepdims=True)
        l2 = jnp.where(iota == a1, -jnp.inf, logits)
        m2 = jnp.max(l2, axis=1, keepdims=True)
        a2 = jnp.min(jnp.where(l2 == m2, iota, _N_EXPERTS), axis=1,
                     keepdims=True)
        e = jnp.exp(m2 - m1)
        g1 = 1.0 / (1.0 + e)
        g2 = e / (1.0 + e)
        gates = (jnp.where(iota == a1, g1, 0.0)
                 + jnp.where(iota == a2, g2, 0.0))
        importance = jnp.sum(gates, axis=0, keepdims=True)
        load = jnp.sum((gates > 0).astype(jnp.float32), axis=0, keepdims=True)

        def cv_sq(v):
            mu = jnp.mean(v)
            var = jnp.sum((v - mu) ** 2) * (1.0 / (_N_EXPERTS - 1))
            return var / (mu * mu + 1e-10)

        loss_ref[0] = (cv_sq(importance) + cv_sq(load)) * _LOSS_COEF
        pi_ref[...] = jnp.where(iota == 0, a1, 0) + jnp.where(iota == 1, a2, 0)
        pf_ref[...] = jnp.where(iota == 0, g1, 0.0) + jnp.where(iota == 1, g2, 0.0)


def _mix_matmul_body(idx_ref, x_ref, w0_ref, w1_ref, b0_ref, b1_ref, g_ref,
                     o_ref, wmix_ref):
    b = pl.program_id(0)
    s = pl.program_id(1)
    g0 = g_ref[b, 0]
    g1 = g_ref[b, 1]

    @pl.when(s == 0)
    def _mix():
        wmix_ref[...] = (g0 * w0_ref[0] + g1 * w1_ref[0]).astype(jnp.bfloat16)

    y = jax.lax.dot_general(
        x_ref[0].astype(jnp.bfloat16), wmix_ref[...],
        (((1,), (1,)), ((), ())),
        preferred_element_type=jnp.float32)
    bm = g0 * b0_ref[0] + g1 * b1_ref[0]
    o_ref[0] = y + bm


def kernel(x, w_gate, weight, bias):
    n_pool = _S // _POOL_BLK
    pack_i, pack_f, loss_arr = pl.pallas_call(
        _gating_body,
        grid=(n_pool,),
        in_specs=[
            pl.BlockSpec((_B, _POOL_BLK, _D_IN), lambda i: (0, i, 0)),
            pl.BlockSpec((_D_IN, _N_EXPERTS), lambda i: (0, 0)),
        ],
        out_specs=[
            pl.BlockSpec((_B, _N_EXPERTS), lambda i: (0, 0)),
            pl.BlockSpec((_B, _N_EXPERTS), lambda i: (0, 0)),
            pl.BlockSpec(memory_space=pltpu.SMEM),
        ],
        out_shape=[
            jax.ShapeDtypeStruct((_B, _N_EXPERTS), jnp.int32),
            jax.ShapeDtypeStruct((_B, _N_EXPERTS), jnp.float32),
            jax.ShapeDtypeStruct((1,), jnp.float32),
        ],
        scratch_shapes=[pltpu.VMEM((_B, _D_IN), jnp.float32)],
    )(x, w_gate)

    idx_flat = pack_i[:, :_TOP_K].reshape(-1)
    gvals = pack_f[:, :_TOP_K]
    loss = loss_arr[0]

    bias3 = bias.reshape(_N_EXPERTS, 1, _D_OUT)
    n_mm = _S // _MM_BLK
    grid_spec = pltpu.PrefetchScalarGridSpec(
        num_scalar_prefetch=1,
        grid=(_B, n_mm),
        in_specs=[
            pl.BlockSpec((1, _MM_BLK, _D_IN), lambda b, s, idx: (b, s, 0)),
            pl.BlockSpec((1, _D_OUT, _D_IN),
                         lambda b, s, idx: (idx[2 * b], 0, 0)),
            pl.BlockSpec((1, _D_OUT, _D_IN),
                         lambda b, s, idx: (idx[2 * b + 1], 0, 0)),
            pl.BlockSpec((1, 1, _D_OUT), lambda b, s, idx: (idx[2 * b], 0, 0)),
            pl.BlockSpec((1, 1, _D_OUT),
                         lambda b, s, idx: (idx[2 * b + 1], 0, 0)),
            pl.BlockSpec(memory_space=pltpu.SMEM),
        ],
        out_specs=pl.BlockSpec((1, _MM_BLK, _D_OUT), lambda b, s, idx: (b, s, 0)),
        scratch_shapes=[pltpu.VMEM((_D_OUT, _D_IN), jnp.bfloat16)],
    )
    y = pl.pallas_call(
        _mix_matmul_body,
        grid_spec=grid_spec,
        out_shape=jax.ShapeDtypeStruct((_B, _S, _D_OUT), jnp.float32),
    )(idx_flat, x, weight, weight, bias3, bias3, gvals)

    return (y, loss)


# speedup vs baseline: 1.0595x; 1.0595x over previous
import jax
import jax.numpy as jnp
from jax.experimental import pallas as pl
from jax.experimental.pallas import tpu as pltpu

_N_EXPERTS = 64
_TOP_K = 2
_D_IN = 768
_D_OUT = 768
_B = 4
_S = 2048
_LOSS_COEF = 0.01

_POOL_BLK = 512
_MM_BLK = 512


def _gating_body(x_ref, wg_ref, b_ref, pi_ref, pf_ref, bm_ref, loss_ref,
                 acc_ref):
    i = pl.program_id(0)
    n = pl.num_programs(0)

    @pl.when(i == 0)
    def _init():
        acc_ref[...] = jnp.zeros_like(acc_ref)

    acc_ref[...] += jnp.sum(x_ref[...], axis=1)

    @pl.when(i == n - 1)
    def _finish():
        pooled = acc_ref[...] * (1.0 / _S)
        logits = jax.lax.dot_general(
            pooled, wg_ref[...], (((1,), (0,)), ((), ())),
            preferred_element_type=jnp.float32)
        iota = jax.lax.broadcasted_iota(jnp.int32, (_B, _N_EXPERTS), 1)
        m1 = jnp.max(logits, axis=1, keepdims=True)
        a1 = jnp.min(jnp.where(logits == m1, iota, _N_EXPERTS), axis=1,
                     keepdims=True)
        l2 = jnp.where(iota == a1, -jnp.inf, logits)
        m2 = jnp.max(l2, axis=1, keepdims=True)
        a2 = jnp.min(jnp.where(l2 == m2, iota, _N_EXPERTS), axis=1,
                     keepdims=True)
        e = jnp.exp(m2 - m1)
        g1 = 1.0 / (1.0 + e)
        g2 = e / (1.0 + e)
        gates = (jnp.where(iota == a1, g1, 0.0)
                 + jnp.where(iota == a2, g2, 0.0))
        importance = jnp.sum(gates, axis=0, keepdims=True)
        load = jnp.sum((gates > 0).astype(jnp.float32), axis=0, keepdims=True)

        def cv_sq(v):
            mu = jnp.mean(v)
            var = jnp.sum((v - mu) ** 2) * (1.0 / (_N_EXPERTS - 1))
            return var / (mu * mu + 1e-10)

        loss_ref[0] = (cv_sq(importance) + cv_sq(load)) * _LOSS_COEF
        pi_ref[...] = jnp.where(iota == 0, a1, 0) + jnp.where(iota == 1, a2, 0)
        pf_ref[...] = jnp.where(iota == 0, g1, 0.0) + jnp.where(iota == 1, g2, 0.0)
        bmix = jax.lax.dot_general(
            gates, b_ref[...], (((1,), (0,)), ((), ())),
            preferred_element_type=jnp.float32)
        bm_ref[...] = bmix[:, None, :]


def _mix_matmul_body(idx_ref, x_ref, w0_ref, w1_ref, bm_ref, g_ref,
                     o_ref, wmix_ref):
    b = pl.program_id(0)
    s = pl.program_id(1)
    g0 = g_ref[b, 0]
    g1 = g_ref[b, 1]

    @pl.when(s == 0)
    def _mix():
        wmix_ref[...] = (g0 * w0_ref[0] + g1 * w1_ref[0]).astype(jnp.bfloat16)

    y = jax.lax.dot_general(
        x_ref[0].astype(jnp.bfloat16), wmix_ref[...],
        (((1,), (1,)), ((), ())),
        preferred_element_type=jnp.float32)
    o_ref[0] = y + bm_ref[0]


def kernel(x, w_gate, weight, bias):
    n_pool = _S // _POOL_BLK
    pack_i, pack_f, bias_mix, loss_arr = pl.pallas_call(
        _gating_body,
        grid=(n_pool,),
        in_specs=[
            pl.BlockSpec((_B, _POOL_BLK, _D_IN), lambda i: (0, i, 0)),
            pl.BlockSpec((_D_IN, _N_EXPERTS), lambda i: (0, 0)),
            pl.BlockSpec((_N_EXPERTS, _D_OUT), lambda i: (0, 0)),
        ],
        out_specs=[
            pl.BlockSpec((_B, _N_EXPERTS), lambda i: (0, 0)),
            pl.BlockSpec((_B, _N_EXPERTS), lambda i: (0, 0)),
            pl.BlockSpec((_B, 1, _D_OUT), lambda i: (0, 0, 0)),
            pl.BlockSpec(memory_space=pltpu.SMEM),
        ],
        out_shape=[
            jax.ShapeDtypeStruct((_B, _N_EXPERTS), jnp.int32),
            jax.ShapeDtypeStruct((_B, _N_EXPERTS), jnp.float32),
            jax.ShapeDtypeStruct((_B, 1, _D_OUT), jnp.float32),
            jax.ShapeDtypeStruct((1,), jnp.float32),
        ],
        scratch_shapes=[pltpu.VMEM((_B, _D_IN), jnp.float32)],
    )(x, w_gate, bias)

    n_mm = _S // _MM_BLK
    grid_spec = pltpu.PrefetchScalarGridSpec(
        num_scalar_prefetch=1,
        grid=(_B, n_mm),
        in_specs=[
            pl.BlockSpec((1, _MM_BLK, _D_IN), lambda b, s, idx: (b, s, 0)),
            pl.BlockSpec((1, _D_OUT, _D_IN),
                         lambda b, s, idx: (idx[b, 0], 0, 0)),
            pl.BlockSpec((1, _D_OUT, _D_IN),
                         lambda b, s, idx: (idx[b, 1], 0, 0)),
            pl.BlockSpec((1, 1, _D_OUT), lambda b, s, idx: (b, 0, 0)),
            pl.BlockSpec(memory_space=pltpu.SMEM),
        ],
        out_specs=pl.BlockSpec((1, _MM_BLK, _D_OUT), lambda b, s, idx: (b, s, 0)),
        scratch_shapes=[pltpu.VMEM((_D_OUT, _D_IN), jnp.bfloat16)],
    )
    y = pl.pallas_call(
        _mix_matmul_body,
        grid_spec=grid_spec,
        out_shape=jax.ShapeDtypeStruct((_B, _S, _D_OUT), jnp.float32),
    )(pack_i, x, weight, weight, bias_mix, pack_f)

    return (y, loss_arr[0])


# speedup vs baseline: 1.4649x; 1.3827x over previous
import jax
import jax.numpy as jnp
from jax.experimental import pallas as pl
from jax.experimental.pallas import tpu as pltpu

_N_EXPERTS = 64
_TOP_K = 2
_D_IN = 768
_D_OUT = 768
_B = 4
_S = 2048
_LOSS_COEF = 0.01

_BLK = 512
_NCH = _S // _BLK


def _fused_body(x_ref, wg_ref, b_ref, w_hbm, o_ref, loss_ref,
                xb_ref, acc_ref, wbuf_ref, wmix_ref, bmix_ref,
                idx_s, g_s, sems):
    i = pl.program_id(0)

    @pl.when(i < _NCH)
    def _pool():
        @pl.when(i == 0)
        def _init():
            acc_ref[...] = jnp.zeros_like(acc_ref)

        acc_ref[...] += jnp.sum(x_ref[...], axis=1)
        xb_ref[i] = x_ref[...].astype(jnp.bfloat16)

    @pl.when(i == _NCH - 1)
    def _gate():
        pooled = acc_ref[...] * (1.0 / _S)
        logits = jax.lax.dot_general(
            pooled, wg_ref[...], (((1,), (0,)), ((), ())),
            preferred_element_type=jnp.float32)
        iota = jax.lax.broadcasted_iota(jnp.int32, (_B, _N_EXPERTS), 1)
        m1 = jnp.max(logits, axis=1, keepdims=True)
        a1 = jnp.min(jnp.where(logits == m1, iota, _N_EXPERTS), axis=1,
                     keepdims=True)
        l2 = jnp.where(iota == a1, -jnp.inf, logits)
        m2 = jnp.max(l2, axis=1, keepdims=True)
        a2 = jnp.min(jnp.where(l2 == m2, iota, _N_EXPERTS), axis=1,
                     keepdims=True)
        e = jnp.exp(m2 - m1)
        g1 = 1.0 / (1.0 + e)
        g2 = e / (1.0 + e)
        gates = (jnp.where(iota == a1, g1, 0.0)
                 + jnp.where(iota == a2, g2, 0.0))
        importance = jnp.sum(gates, axis=0, keepdims=True)
        load = jnp.sum((gates > 0).astype(jnp.float32), axis=0, keepdims=True)

        def cv_sq(v):
            mu = jnp.mean(v)
            var = jnp.sum((v - mu) ** 2) * (1.0 / (_N_EXPERTS - 1))
            return var / (mu * mu + 1e-10)

        loss_ref[0] = (cv_sq(importance) + cv_sq(load)) * _LOSS_COEF
        bmix = jax.lax.dot_general(
            gates, b_ref[...], (((1,), (0,)), ((), ())),
            preferred_element_type=jnp.float32)
        bmix_ref[...] = bmix[:, None, :]

        for bb in range(_B):
            i1 = jnp.max(jnp.where(a1[bb:bb + 1, :] < _N_EXPERTS,
                                   a1[bb:bb + 1, :], 0))
            i2 = jnp.max(jnp.where(a2[bb:bb + 1, :] < _N_EXPERTS,
                                   a2[bb:bb + 1, :], 0))
            idx_s[2 * bb] = i1
            idx_s[2 * bb + 1] = i2
            g_s[2 * bb] = jnp.max(g1[bb:bb + 1, :])
            g_s[2 * bb + 1] = jnp.max(g2[bb:bb + 1, :])
            pltpu.make_async_copy(
                w_hbm.at[i1], wbuf_ref.at[bb, 0],
                sems.at[bb, 0]).start()
            pltpu.make_async_copy(
                w_hbm.at[i2], wbuf_ref.at[bb, 1],
                sems.at[bb, 1]).start()

    @pl.when(i >= _NCH)
    def _matmul():
        j = i - _NCH
        b = j // _NCH
        s = j % _NCH

        @pl.when(s == 0)
        def _mix():
            pltpu.make_async_copy(
                w_hbm.at[0], wbuf_ref.at[b, 0], sems.at[b, 0]).wait()
            pltpu.make_async_copy(
                w_hbm.at[0], wbuf_ref.at[b, 1], sems.at[b, 1]).wait()
            g0 = g_s[2 * b]
            g1v = g_s[2 * b + 1]
            wmix_ref[...] = (g0 * wbuf_ref[b, 0]
                             + g1v * wbuf_ref[b, 1]).astype(jnp.bfloat16)

        y = jax.lax.dot_general(
            xb_ref[s, b], wmix_ref[...], (((1,), (1,)), ((), ())),
            preferred_element_type=jnp.float32)
        o_ref[0] = y + bmix_ref[b]


def kernel(x, w_gate, weight, bias):
    def _x_map(i):
        return (0, jnp.minimum(i, _NCH - 1), 0)

    def _o_map(i):
        j = jnp.maximum(i - _NCH, 0)
        return (j // _NCH, j % _NCH, 0)

    y, loss_arr = pl.pallas_call(
        _fused_body,
        grid=(_NCH + _B * _NCH,),
        in_specs=[
            pl.BlockSpec((_B, _BLK, _D_IN), _x_map),
            pl.BlockSpec((_D_IN, _N_EXPERTS), lambda i: (0, 0)),
            pl.BlockSpec((_N_EXPERTS, _D_OUT), lambda i: (0, 0)),
            pl.BlockSpec(memory_space=pl.ANY),
        ],
        out_specs=[
            pl.BlockSpec((1, _BLK, _D_OUT), _o_map),
            pl.BlockSpec(memory_space=pltpu.SMEM),
        ],
        out_shape=[
            jax.ShapeDtypeStruct((_B, _S, _D_OUT), jnp.float32),
            jax.ShapeDtypeStruct((1,), jnp.float32),
        ],
        scratch_shapes=[
            pltpu.VMEM((_NCH, _B, _BLK, _D_IN), jnp.bfloat16),
            pltpu.VMEM((_B, _D_IN), jnp.float32),
            pltpu.VMEM((_B, _TOP_K, _D_OUT, _D_IN), jnp.float32),
            pltpu.VMEM((_D_OUT, _D_IN), jnp.bfloat16),
            pltpu.VMEM((_B, 1, _D_OUT), jnp.float32),
            pltpu.SMEM((2 * _B,), jnp.int32),
            pltpu.SMEM((2 * _B,), jnp.float32),
            pltpu.SemaphoreType.DMA((_B, _TOP_K)),
        ],
    )(x, w_gate, bias, weight)

    return (y, loss_arr[0])
